# Initial kernel scaffold; baseline (speedup 1.0000x reference)
import jax
import jax.numpy as jnp
from jax.experimental import pallas as pl


def kernel(reference, attr, attention, batch_index):
    raise NotImplementedError("write your pallas kernel here")


# speedup vs baseline: 1.0861x; 1.0861x over previous
import functools

import jax
import jax.numpy as jnp
from jax import lax
from jax.experimental import pallas as pl
from jax.experimental.pallas import tpu as pltpu
from jax.experimental.pallas import tpu_sc as plsc

N = 100000
D = 512
G = 256
NC = 2
NS = 16
L = 16
NW = NC * NS

CH = 32
NCHUNK = N // CH
BIGW = NCHUNK % NW
CPW = NCHUNK // NW
BIG_ROWS = (CPW + 1) * CH
SMALL_ROWS = CPW * CH
BIG_BASE = BIGW * BIG_ROWS

SEG_PER_W = G // NW

_SPLAT_DNUMS = lax.GatherDimensionNumbers(
    offset_dims=(), collapsed_slice_dims=(0,), start_index_map=(0,))


def _splat(vec, lane):
    idx = jnp.full((L, 1), lane, jnp.int32)
    return lax.gather(vec, idx, _SPLAT_DNUMS, slice_sizes=(1,),
                      mode=lax.GatherScatterMode.PROMISE_IN_BOUNDS)


_mesh = functools.partial(
    plsc.VectorSubcoreMesh,
    core_axis_name="c", subcore_axis_name="s", num_cores=NC, num_subcores=NS)

_params = pltpu.CompilerParams(needs_layout_passes=False)

@functools.partial(
    pl.kernel,
    out_type=[
        jax.ShapeDtypeStruct((NW * G * D,), jnp.float32),
        jax.ShapeDtypeStruct((NW * G * L,), jnp.float32),
    ],
    mesh=_mesh(),
    compiler_params=_params,
    scratch_types=[
        pltpu.VMEM((CH, D), jnp.float32),
        pltpu.VMEM((CH,), jnp.float32),
        pltpu.VMEM((CH,), jnp.int32),
        pltpu.VMEM((D,), jnp.float32),
        pltpu.VMEM((L,), jnp.float32),
        pltpu.VMEM((L * D,), jnp.float32),
        pltpu.VMEM((L * L,), jnp.float32),
    ],
)
def _pool(attr_h, att_h, bi_h, acc_out, den_out,
          rows_v, att_v, bi_v, acc_v, accw_v, zrows_v, zden_v):
    cid = lax.axis_index("c")
    sid = lax.axis_index("s")
    wid = sid * NC + cid

    nch = jnp.where(wid < BIGW, CPW + 1, CPW)
    wbase = jnp.where(wid < BIGW, wid * BIG_ROWS,
                      BIG_BASE + (wid - BIGW) * SMALL_ROWS)

    zero = jnp.zeros((L,), jnp.float32)

    for j in range(D // L):
        acc_v[pl.ds(j * L, L)] = zero
    accw_v[...] = zero

    def _zero_blk(n, _):
        for j in range(D // L):
            zrows_v[pl.ds(n * D + j * L, L)] = zero
        zden_v[pl.ds(n * L, L)] = zero
        return 0
    lax.fori_loop(0, L, _zero_blk, 0)

    def _zero_slab(z, _):
        pltpu.sync_copy(zrows_v, acc_out.at[pl.ds((wid * G + z * L) * D, L * D)])
        pltpu.sync_copy(zden_v, den_out.at[pl.ds((wid * G + z * L) * L, L * L)])
        return 0
    lax.fori_loop(0, G // L, _zero_slab, 0)

    def _flush(seg):
        pltpu.sync_copy(acc_v, acc_out.at[pl.ds((wid * G + seg) * D, D)])
        pltpu.sync_copy(accw_v, den_out.at[pl.ds((wid * G + seg) * L, L)])
        for j in range(D // L):
            acc_v[pl.ds(j * L, L)] = zero
        accw_v[...] = zero

    pltpu.sync_copy(bi_h.at[pl.ds(wbase, L)], bi_v.at[pl.ds(0, L)])
    prev0 = bi_v[pl.ds(0, L)][0]

    def _chunk(c, prev):
        gbase = wbase + c * CH
        pltpu.sync_copy(attr_h.at[pl.ds(gbase, CH)], rows_v)
        pltpu.sync_copy(att_h.at[pl.ds(gbase, CH)], att_v)
        pltpu.sync_copy(bi_h.at[pl.ds(gbase, CH)], bi_v)
        for gi in range(CH // L):
            biv = bi_v[pl.ds(gi * L, L)]
            wv = jnp.exp(att_v[pl.ds(gi * L, L)])
            for lane in range(L):
                r = gi * L + lane
                s = biv[lane]

                @pl.when(s != prev)
                def _():
                    _flush(prev)

                prev = s
                w = _splat(wv, lane)
                plsc.addupdate(accw_v.at[pl.ds(0, L)], w)
                for j in range(D // L):
                    plsc.addupdate(acc_v.at[pl.ds(j * L, L)],
                                   rows_v[r, pl.ds(j * L, L)] * w)
        return prev

    prev = lax.fori_loop(0, nch, _chunk, prev0)
    _flush(prev)


@functools.partial(
    pl.kernel,
    out_type=jax.ShapeDtypeStruct((G, D), jnp.float32),
    mesh=_mesh(),
    compiler_params=_params,
    scratch_types=[
        pltpu.VMEM((SEG_PER_W * D,), jnp.float32),
        pltpu.VMEM((SEG_PER_W * L,), jnp.float32),
        pltpu.VMEM((SEG_PER_W * D,), jnp.float32),
        pltpu.VMEM((SEG_PER_W * L,), jnp.float32),
        pltpu.VMEM((SEG_PER_W, D), jnp.float32),
    ],
)
def _combine(acc_h, den_h, out_h, slab_v, dslab_v, sum_v, wsum_v, ob):
    cid = lax.axis_index("c")
    sid = lax.axis_index("s")
    wid = sid * NC + cid
    sbase = wid * SEG_PER_W

    zero = jnp.zeros((L,), jnp.float32)
    for g in range(SEG_PER_W):
        for j in range(D // L):
            sum_v[pl.ds(g * D + j * L, L)] = zero
        wsum_v[pl.ds(g * L, L)] = zero

    def _accum(w2, _):
        pltpu.sync_copy(acc_h.at[pl.ds((w2 * G + sbase) * D, SEG_PER_W * D)],
                        slab_v)
        pltpu.sync_copy(den_h.at[pl.ds((w2 * G + sbase) * L, SEG_PER_W * L)],
                        dslab_v)
        for g in range(SEG_PER_W):
            for j in range(D // L):
                plsc.addupdate(sum_v.at[pl.ds(g * D + j * L, L)],
                               slab_v[pl.ds(g * D + j * L, L)])
            plsc.addupdate(wsum_v.at[pl.ds(g * L, L)], dslab_v[pl.ds(g * L, L)])
        return 0
    lax.fori_loop(0, NW, _accum, 0)

    for g in range(SEG_PER_W):
        dv = wsum_v[pl.ds(g * L, L)]
        nonempty = dv > 0.0
        scale = jnp.where(nonempty, 1.0 / jnp.where(nonempty, dv, 1.0), 0.0)
        for j in range(D // L):
            ob[g, pl.ds(j * L, L)] = sum_v[pl.ds(g * D + j * L, L)] * scale
    pltpu.sync_copy(ob, out_h.at[pl.ds(sbase, SEG_PER_W)])


def kernel(reference, attr, attention, batch_index):
    del reference
    att = attention.reshape((N,))
    bi = batch_index.astype(jnp.int32)
    acc, den = _pool(attr, att, bi)
    return _combine(acc, den)


# speedup vs baseline: 2.8738x; 2.6459x over previous
import functools

import jax
import jax.numpy as jnp
from jax import lax
from jax.experimental import pallas as pl
from jax.experimental.pallas import tpu as pltpu
from jax.experimental.pallas import tpu_sc as plsc

N = 100000
D = 512
G = 256
NC = 2
NS = 16
L = 16
NW = NC * NS
NJ = D // L

CH = L
NCHUNK = N // CH
BIGW = (NCHUNK // 2) % NW
PPW = NCHUNK // 2 // NW
BIG_ROWS = (PPW + 1) * 2 * CH
SMALL_ROWS = PPW * 2 * CH
BIG_BASE = BIGW * BIG_ROWS

SEG_PER_W = G // NW

_SPLAT_DNUMS = lax.GatherDimensionNumbers(
    offset_dims=(), collapsed_slice_dims=(0,), start_index_map=(0,))


def _vgather(vec, idx):
    return lax.gather(vec, idx.reshape(L, 1), _SPLAT_DNUMS, slice_sizes=(1,),
                      mode=lax.GatherScatterMode.PROMISE_IN_BOUNDS)


def _splat(vec, lane):
    return _vgather(vec, jnp.full((L,), lane, jnp.int32))


_mesh = functools.partial(
    plsc.VectorSubcoreMesh,
    core_axis_name="c", subcore_axis_name="s", num_cores=NC, num_subcores=NS)

_params = pltpu.CompilerParams(needs_layout_passes=False)


@functools.partial(
    pl.kernel,
    out_type=[
        jax.ShapeDtypeStruct((NW * G * D,), jnp.float32),
        jax.ShapeDtypeStruct((NW * G * L,), jnp.float32),
    ],
    mesh=_mesh(),
    compiler_params=_params,
    scratch_types=[
        pltpu.VMEM((CH, D), jnp.float32),
        pltpu.VMEM((CH, D), jnp.float32),
        pltpu.VMEM((BIG_ROWS,), jnp.float32),
        pltpu.VMEM((BIG_ROWS + L,), jnp.int32),
        pltpu.VMEM((D,), jnp.float32),
        pltpu.VMEM((L,), jnp.float32),
        pltpu.VMEM((L * D,), jnp.float32),
        pltpu.VMEM((L * L,), jnp.float32),
        pltpu.SemaphoreType.DMA,
        pltpu.SemaphoreType.DMA,
        pltpu.SemaphoreType.DMA,
    ],
)
def _pool(attr_h, att_h, bi_h, acc_out, den_out,
          rows_a, rows_b, att_v, bi_v, acc_v, accw_v, zrows_v, zden_v,
          sem_a, sem_b, sem_z):
    cid = lax.axis_index("c")
    sid = lax.axis_index("s")
    wid = sid * NC + cid

    npairs = jnp.where(wid < BIGW, PPW + 1, PPW)
    wbase = jnp.where(wid < BIGW, wid * BIG_ROWS,
                      BIG_BASE + (wid - BIGW) * SMALL_ROWS)

    zero = jnp.zeros((L,), jnp.float32)

    for j in range(NJ):
        acc_v[pl.ds(j * L, L)] = zero
    accw_v[...] = zero

    def _zero_blk(n, _):
        for j in range(NJ):
            zrows_v[pl.ds(n * D + j * L, L)] = zero
        zden_v[pl.ds(n * L, L)] = zero
        return 0
    lax.fori_loop(0, L, _zero_blk, 0)

    def _zs_issue(z, _):
        pltpu.async_copy(zrows_v, acc_out.at[pl.ds((wid * G + z * L) * D, L * D)],
                         sem_z)
        pltpu.async_copy(zden_v, den_out.at[pl.ds((wid * G + z * L) * L, L * L)],
                         sem_z)
        return 0
    lax.fori_loop(0, G // L, _zs_issue, 0)

    @pl.when(wid < BIGW)
    def _():
        pltpu.sync_copy(att_h.at[pl.ds(wbase, BIG_ROWS)],
                        att_v.at[pl.ds(0, BIG_ROWS)])
        pltpu.sync_copy(bi_h.at[pl.ds(wbase, BIG_ROWS)],
                        bi_v.at[pl.ds(0, BIG_ROWS)])

    @pl.when(wid >= BIGW)
    def _():
        pltpu.sync_copy(att_h.at[pl.ds(wbase, SMALL_ROWS)],
                        att_v.at[pl.ds(0, SMALL_ROWS)])
        pltpu.sync_copy(bi_h.at[pl.ds(wbase, SMALL_ROWS)],
                        bi_v.at[pl.ds(0, SMALL_ROWS)])

    def _zs_drain(z, _):
        pltpu.make_async_copy(
            zrows_v, acc_out.at[pl.ds((wid * G + z * L) * D, L * D)], sem_z
        ).wait()
        pltpu.make_async_copy(
            zden_v, den_out.at[pl.ds((wid * G + z * L) * L, L * L)], sem_z
        ).wait()
        return 0
    lax.fori_loop(0, G // L, _zs_drain, 0)

    def _flush(seg):
        pltpu.sync_copy(acc_v, acc_out.at[pl.ds((wid * G + seg) * D, D)])
        pltpu.sync_copy(accw_v, den_out.at[pl.ds((wid * G + seg) * L, L)])
        for j in range(NJ):
            acc_v[pl.ds(j * L, L)] = zero
        accw_v[...] = zero

    lanes = lax.iota(jnp.int32, L)
    shift_idx = jnp.maximum(lanes - 1, 0)

    def _process(buf, lbase, prev):
        biv = bi_v[pl.ds(lbase, L)]
        wv = jnp.exp(att_v[pl.ds(lbase, L)])
        shifted = _vgather(biv, shift_idx)
        shifted = jnp.where(lanes == 0, jnp.full((L,), prev, jnp.int32), shifted)
        has_boundary = jnp.any(shifted != biv)

        @pl.when(jnp.logical_not(has_boundary))
        def _():
            w = _splat(wv, 0)
            wsum = w
            regs = [buf[0, pl.ds(j * L, L)] * w for j in range(NJ)]
            for r in range(1, L):
                w = _splat(wv, r)
                wsum = wsum + w
                for j in range(NJ):
                    regs[j] = regs[j] + buf[r, pl.ds(j * L, L)] * w
            for j in range(NJ):
                plsc.addupdate(acc_v.at[pl.ds(j * L, L)], regs[j])
            plsc.addupdate(accw_v.at[pl.ds(0, L)], wsum)

        @pl.when(has_boundary)
        def _():
            def lane_body(li, prevl):
                s = bi_v[pl.ds(lbase + li, L)][0]

                @pl.when(s != prevl)
                def _():
                    _flush(prevl)

                w = _vgather(wv, jnp.full((L,), li, jnp.int32))
                plsc.addupdate(accw_v.at[pl.ds(0, L)], w)
                for j in range(NJ):
                    plsc.addupdate(acc_v.at[pl.ds(j * L, L)],
                                   buf[li, pl.ds(j * L, L)] * w)
                return s
            lax.fori_loop(0, L, lane_body, prev)

        return biv[L - 1]

    prev0 = bi_v[pl.ds(0, L)][0]

    pltpu.async_copy(attr_h.at[pl.ds(wbase, CH)], rows_a, sem_a)

    def _pair(p, prev):
        base0 = wbase + p * 2 * CH
        base1 = base0 + CH
        pltpu.async_copy(attr_h.at[pl.ds(base1, CH)], rows_b, sem_b)
        pltpu.make_async_copy(attr_h.at[pl.ds(base0, CH)], rows_a, sem_a).wait()
        prev = _process(rows_a, p * 2 * CH, prev)

        @pl.when(p + 1 < npairs)
        def _():
            pltpu.async_copy(attr_h.at[pl.ds(base1 + CH, CH)], rows_a, sem_a)

        pltpu.make_async_copy(attr_h.at[pl.ds(base1, CH)], rows_b, sem_b).wait()
        prev = _process(rows_b, p * 2 * CH + CH, prev)
        return prev

    prev = lax.fori_loop(0, npairs, _pair, prev0)
    _flush(prev)


@functools.partial(
    pl.kernel,
    out_type=jax.ShapeDtypeStruct((G, D), jnp.float32),
    mesh=_mesh(),
    compiler_params=_params,
    scratch_types=[
        pltpu.VMEM((SEG_PER_W * D,), jnp.float32),
        pltpu.VMEM((SEG_PER_W * L,), jnp.float32),
        pltpu.VMEM((SEG_PER_W * D,), jnp.float32),
        pltpu.VMEM((SEG_PER_W * L,), jnp.float32),
        pltpu.VMEM((SEG_PER_W, D), jnp.float32),
    ],
)
def _combine(acc_h, den_h, out_h, slab_v, dslab_v, sum_v, wsum_v, ob):
    cid = lax.axis_index("c")
    sid = lax.axis_index("s")
    wid = sid * NC + cid
    sbase = wid * SEG_PER_W

    zero = jnp.zeros((L,), jnp.float32)
    for g in range(SEG_PER_W):
        for j in range(NJ):
            sum_v[pl.ds(g * D + j * L, L)] = zero
        wsum_v[pl.ds(g * L, L)] = zero

    def _accum(w2, _):
        pltpu.sync_copy(acc_h.at[pl.ds((w2 * G + sbase) * D, SEG_PER_W * D)],
                        slab_v)
        pltpu.sync_copy(den_h.at[pl.ds((w2 * G + sbase) * L, SEG_PER_W * L)],
                        dslab_v)
        for g in range(SEG_PER_W):
            for j in range(NJ):
                plsc.addupdate(sum_v.at[pl.ds(g * D + j * L, L)],
                               slab_v[pl.ds(g * D + j * L, L)])
            plsc.addupdate(wsum_v.at[pl.ds(g * L, L)], dslab_v[pl.ds(g * L, L)])
        return 0
    lax.fori_loop(0, NW, _accum, 0)

    for g in range(SEG_PER_W):
        dv = wsum_v[pl.ds(g * L, L)]
        nonempty = dv > 0.0
        scale = jnp.where(nonempty, 1.0 / jnp.where(nonempty, dv, 1.0), 0.0)
        for j in range(NJ):
            ob[g, pl.ds(j * L, L)] = sum_v[pl.ds(g * D + j * L, L)] * scale
    pltpu.sync_copy(ob, out_h.at[pl.ds(sbase, SEG_PER_W)])


def kernel(reference, attr, attention, batch_index):
    del reference
    att = attention.reshape((N,))
    bi = batch_index.astype(jnp.int32)
    acc, den = _pool(attr, att, bi)
    return _combine(acc, den)


# speedup vs baseline: 6.2077x; 2.1601x over previous
import functools

import jax
import jax.numpy as jnp
from jax import lax
from jax.experimental import pallas as pl
from jax.experimental.pallas import tpu as pltpu
from jax.experimental.pallas import tpu_sc as plsc

N = 100000
D = 512
G = 256
NC = 2
NS = 16
L = 16
NW = NC * NS
NJ = D // L

CH = L
NCHUNK = N // CH
BIGW = (NCHUNK // 2) % NW
PPW = NCHUNK // 2 // NW
BIG_ROWS = (PPW + 1) * 2 * CH
SMALL_ROWS = PPW * 2 * CH
BIG_BASE = BIGW * BIG_ROWS

SEG_PER_W = G // NW

_SPLAT_DNUMS = lax.GatherDimensionNumbers(
    offset_dims=(), collapsed_slice_dims=(0,), start_index_map=(0,))


def _vgather(vec, idx):
    return lax.gather(vec, idx.reshape(L, 1), _SPLAT_DNUMS, slice_sizes=(1,),
                      mode=lax.GatherScatterMode.PROMISE_IN_BOUNDS)


def _splat(vec, lane):
    return _vgather(vec, jnp.full((L,), lane, jnp.int32))


_mesh = functools.partial(
    plsc.VectorSubcoreMesh,
    core_axis_name="c", subcore_axis_name="s", num_cores=NC, num_subcores=NS)

_params = pltpu.CompilerParams(needs_layout_passes=False)


@functools.partial(
    pl.kernel,
    out_type=[
        jax.ShapeDtypeStruct((NW * G * D,), jnp.float32),
        jax.ShapeDtypeStruct((NW * G * L,), jnp.float32),
    ],
    mesh=_mesh(),
    compiler_params=_params,
    scratch_types=[
        pltpu.VMEM((CH, D), jnp.float32),
        pltpu.VMEM((CH, D), jnp.float32),
        pltpu.VMEM((BIG_ROWS,), jnp.float32),
        pltpu.VMEM((BIG_ROWS + L,), jnp.int32),
        pltpu.VMEM((D,), jnp.float32),
        pltpu.VMEM((L,), jnp.float32),
        pltpu.VMEM((L * D,), jnp.float32),
        pltpu.VMEM((L * L,), jnp.float32),
        pltpu.SemaphoreType.DMA,
        pltpu.SemaphoreType.DMA,
        pltpu.SemaphoreType.DMA,
    ],
)
def _pool(attr_h, att_h, bi_h, acc_out, den_out,
          rows_a, rows_b, att_v, bi_v, acc_v, accw_v, zrows_v, zden_v,
          sem_a, sem_b, sem_z):
    cid = lax.axis_index("c")
    sid = lax.axis_index("s")
    wid = sid * NC + cid

    npairs = jnp.where(wid < BIGW, PPW + 1, PPW)
    wbase = jnp.where(wid < BIGW, wid * BIG_ROWS,
                      BIG_BASE + (wid - BIGW) * SMALL_ROWS)

    zero = jnp.zeros((L,), jnp.float32)

    for j in range(NJ):
        acc_v[pl.ds(j * L, L)] = zero
    accw_v[...] = zero

    def _zero_blk(n, _):
        for j in range(NJ):
            zrows_v[pl.ds(n * D + j * L, L)] = zero
        zden_v[pl.ds(n * L, L)] = zero
        return 0
    lax.fori_loop(0, L, _zero_blk, 0)

    def _zs_issue(z, _):
        pltpu.async_copy(zrows_v, acc_out.at[pl.ds((wid * G + z * L) * D, L * D)],
                         sem_z)
        pltpu.async_copy(zden_v, den_out.at[pl.ds((wid * G + z * L) * L, L * L)],
                         sem_z)
        return 0
    lax.fori_loop(0, G // L, _zs_issue, 0)

    @pl.when(wid < BIGW)
    def _():
        pltpu.sync_copy(att_h.at[pl.ds(wbase, BIG_ROWS)],
                        att_v.at[pl.ds(0, BIG_ROWS)])
        pltpu.sync_copy(bi_h.at[pl.ds(wbase, BIG_ROWS)],
                        bi_v.at[pl.ds(0, BIG_ROWS)])

    @pl.when(wid >= BIGW)
    def _():
        pltpu.sync_copy(att_h.at[pl.ds(wbase, SMALL_ROWS)],
                        att_v.at[pl.ds(0, SMALL_ROWS)])
        pltpu.sync_copy(bi_h.at[pl.ds(wbase, SMALL_ROWS)],
                        bi_v.at[pl.ds(0, SMALL_ROWS)])

    def _zs_drain(z, _):
        pltpu.make_async_copy(
            zrows_v, acc_out.at[pl.ds((wid * G + z * L) * D, L * D)], sem_z
        ).wait()
        pltpu.make_async_copy(
            zden_v, den_out.at[pl.ds((wid * G + z * L) * L, L * L)], sem_z
        ).wait()
        return 0
    lax.fori_loop(0, G // L, _zs_drain, 0)

    def _flush(seg):
        pltpu.sync_copy(acc_v, acc_out.at[pl.ds((wid * G + seg) * D, D)])
        pltpu.sync_copy(accw_v, den_out.at[pl.ds((wid * G + seg) * L, L)])
        for j in range(NJ):
            acc_v[pl.ds(j * L, L)] = zero
        accw_v[...] = zero

    lanes = lax.iota(jnp.int32, L)
    shift_idx = jnp.maximum(lanes - 1, 0)

    def _process(buf, lbase, prev):
        biv = bi_v[pl.ds(lbase, L)]
        wv = jnp.exp(att_v[pl.ds(lbase, L)])
        shifted = _vgather(biv, shift_idx)
        shifted = jnp.where(lanes == 0, jnp.full((L,), prev, jnp.int32), shifted)
        has_boundary = jnp.any(shifted != biv)

        @pl.when(jnp.logical_not(has_boundary))
        def _():
            wsum = _splat(wv, 0)
            for r in range(1, L):
                wsum = wsum + _splat(wv, r)
            plsc.addupdate(accw_v.at[pl.ds(0, L)], wsum)
            for jt in range(0, NJ, 8):
                w = _splat(wv, 0)
                regs = [buf[0, pl.ds((jt + j) * L, L)] * w for j in range(8)]
                for r in range(1, L):
                    w = _splat(wv, r)
                    for j in range(8):
                        regs[j] = regs[j] + buf[r, pl.ds((jt + j) * L, L)] * w
                for j in range(8):
                    plsc.addupdate(acc_v.at[pl.ds((jt + j) * L, L)], regs[j])

        @pl.when(has_boundary)
        def _():
            def lane_body(li, prevl):
                s = bi_v[pl.ds(lbase + li, L)][0]

                @pl.when(s != prevl)
                def _():
                    _flush(prevl)

                w = _vgather(wv, jnp.full((L,), li, jnp.int32))
                plsc.addupdate(accw_v.at[pl.ds(0, L)], w)
                for j in range(NJ):
                    plsc.addupdate(acc_v.at[pl.ds(j * L, L)],
                                   buf[li, pl.ds(j * L, L)] * w)
                return s
            lax.fori_loop(0, L, lane_body, prev)

        return biv[L - 1]

    prev0 = bi_v[pl.ds(0, L)][0]

    pltpu.async_copy(attr_h.at[pl.ds(wbase, CH)], rows_a, sem_a)

    def _pair(p, prev):
        base0 = wbase + p * 2 * CH
        base1 = base0 + CH
        pltpu.async_copy(attr_h.at[pl.ds(base1, CH)], rows_b, sem_b)
        pltpu.make_async_copy(attr_h.at[pl.ds(base0, CH)], rows_a, sem_a).wait()
        prev = _process(rows_a, p * 2 * CH, prev)

        @pl.when(p + 1 < npairs)
        def _():
            pltpu.async_copy(attr_h.at[pl.ds(base1 + CH, CH)], rows_a, sem_a)

        pltpu.make_async_copy(attr_h.at[pl.ds(base1, CH)], rows_b, sem_b).wait()
        prev = _process(rows_b, p * 2 * CH + CH, prev)
        return prev

    prev = lax.fori_loop(0, npairs, _pair, prev0)
    _flush(prev)


@functools.partial(
    pl.kernel,
    out_type=jax.ShapeDtypeStruct((G, D), jnp.float32),
    mesh=_mesh(),
    compiler_params=_params,
    scratch_types=[
        pltpu.VMEM((SEG_PER_W * D,), jnp.float32),
        pltpu.VMEM((SEG_PER_W * D,), jnp.float32),
        pltpu.VMEM((SEG_PER_W * L,), jnp.float32),
        pltpu.VMEM((SEG_PER_W * L,), jnp.float32),
        pltpu.VMEM((SEG_PER_W * D,), jnp.float32),
        pltpu.VMEM((SEG_PER_W * L,), jnp.float32),
        pltpu.VMEM((SEG_PER_W, D), jnp.float32),
        pltpu.SemaphoreType.DMA,
        pltpu.SemaphoreType.DMA,
    ],
)
def _combine(acc_h, den_h, out_h, slab_a, slab_b, dslab_a, dslab_b,
             sum_v, wsum_v, ob, sem_a, sem_b):
    cid = lax.axis_index("c")
    sid = lax.axis_index("s")
    wid = sid * NC + cid
    sbase = wid * SEG_PER_W

    zero = jnp.zeros((L,), jnp.float32)
    for g in range(SEG_PER_W):
        for j in range(NJ):
            sum_v[pl.ds(g * D + j * L, L)] = zero
        wsum_v[pl.ds(g * L, L)] = zero

    def _issue(w2, slab, dslab, sem):
        pltpu.async_copy(acc_h.at[pl.ds((w2 * G + sbase) * D, SEG_PER_W * D)],
                         slab, sem)
        pltpu.async_copy(den_h.at[pl.ds((w2 * G + sbase) * L, SEG_PER_W * L)],
                         dslab, sem)

    def _wait(w2, slab, dslab, sem):
        pltpu.make_async_copy(
            acc_h.at[pl.ds((w2 * G + sbase) * D, SEG_PER_W * D)], slab, sem
        ).wait()
        pltpu.make_async_copy(
            den_h.at[pl.ds((w2 * G + sbase) * L, SEG_PER_W * L)], dslab, sem
        ).wait()

    def _add(slab, dslab):
        for g in range(SEG_PER_W):
            for j in range(NJ):
                plsc.addupdate(sum_v.at[pl.ds(g * D + j * L, L)],
                               slab[pl.ds(g * D + j * L, L)])
            plsc.addupdate(wsum_v.at[pl.ds(g * L, L)], dslab[pl.ds(g * L, L)])

    _issue(0, slab_a, dslab_a, sem_a)

    def _accum(p, _):
        w2 = p * 2
        _issue(w2 + 1, slab_b, dslab_b, sem_b)
        _wait(w2, slab_a, dslab_a, sem_a)
        _add(slab_a, dslab_a)

        @pl.when(p + 1 < NW // 2)
        def _():
            _issue(w2 + 2, slab_a, dslab_a, sem_a)

        _wait(w2 + 1, slab_b, dslab_b, sem_b)
        _add(slab_b, dslab_b)
        return 0
    lax.fori_loop(0, NW // 2, _accum, 0)

    for g in range(SEG_PER_W):
        dv = wsum_v[pl.ds(g * L, L)]
        nonempty = dv > 0.0
        scale = jnp.where(nonempty, 1.0 / jnp.where(nonempty, dv, 1.0), 0.0)
        for j in range(NJ):
            ob[g, pl.ds(j * L, L)] = sum_v[pl.ds(g * D + j * L, L)] * scale
    pltpu.sync_copy(ob, out_h.at[pl.ds(sbase, SEG_PER_W)])


def kernel(reference, attr, attention, batch_index):
    del reference
    att = attention.reshape((N,))
    bi = batch_index.astype(jnp.int32)
    acc, den = _pool(attr, att, bi)
    return _combine(acc, den)
